# baseline (device time: 79396 ns/iter reference)
import jax
import jax.numpy as jnp
from jax import lax
from jax.experimental import pallas as pl
from jax.experimental.pallas import tpu as pltpu

N_DEV = 8
MASKS = (1, 3, 4)


def kernel(x, Win0, Wout0, Win1, Wout1, Win2, Wout2):
    b, d = x.shape

    def body(x_ref, win0, wout0, win1, wout1, win2, wout2, out_ref,
             sbuf, rbuf, send_sems, recv_sems):
        my = lax.axis_index("i")

        barrier = pltpu.get_barrier_semaphore()
        for mask in MASKS:
            pl.semaphore_signal(
                barrier, inc=1,
                device_id=(my ^ mask,), device_id_type=pl.DeviceIdType.MESH,
            )
        pl.semaphore_wait(barrier, len(MASKS))

        wins = (win0, win1, win2)
        wouts = (wout0, wout1, wout2)

        x_val = x_ref[...]
        for l in range(3):
            h = jnp.maximum(
                jnp.dot(x_val, wins[l][...], preferred_element_type=jnp.float32),
                0.0,
            )
            acc = jnp.dot(h, wouts[l][...], preferred_element_type=jnp.float32)
            for r, mask in enumerate(MASKS):
                sbuf[l, r] = acc
                rdma = pltpu.make_async_remote_copy(
                    src_ref=sbuf.at[l, r],
                    dst_ref=rbuf.at[l, r],
                    send_sem=send_sems.at[l, r],
                    recv_sem=recv_sems.at[l, r],
                    device_id=(my ^ mask,),
                    device_id_type=pl.DeviceIdType.MESH,
                )
                rdma.start()
                rdma.wait()
                acc = acc + rbuf[l, r]
            x_val = acc
        out_ref[...] = x_val

    return pl.pallas_call(
        body,
        out_shape=jax.ShapeDtypeStruct((b, d), jnp.float32),
        in_specs=[pl.BlockSpec(memory_space=pltpu.VMEM)] * 7,
        out_specs=pl.BlockSpec(memory_space=pltpu.VMEM),
        scratch_shapes=[
            pltpu.VMEM((3, 3, b, d), jnp.float32),
            pltpu.VMEM((3, 3, b, d), jnp.float32),
            pltpu.SemaphoreType.DMA((3, 3)),
            pltpu.SemaphoreType.DMA((3, 3)),
        ],
        compiler_params=pltpu.CompilerParams(collective_id=0),
    )(x, Win0, Wout0, Win1, Wout1, Win2, Wout2)


# device time: 46793 ns/iter; 1.6967x vs baseline; 1.6967x over previous
import jax
import jax.numpy as jnp
from jax import lax
from jax.experimental import pallas as pl
from jax.experimental.pallas import tpu as pltpu

N_DEV = 8
MASKS = (1, 3, 4)
CHUNKS = (176, 168, 168)
OFFS = (0, 176, 344)


def kernel(x, Win0, Wout0, Win1, Wout1, Win2, Wout2):
    b, d = x.shape

    def body(x_ref, win0, wout0, win1, wout1, win2, wout2, out_ref,
             sb0, sb1, sb2, rb0, rb1, rb2, send_sems, recv_sems):
        my = lax.axis_index("i")
        sbufs = (sb0, sb1, sb2)
        rbufs = (rb0, rb1, rb2)

        barrier = pltpu.get_barrier_semaphore()
        for mask in MASKS:
            pl.semaphore_signal(
                barrier, inc=1,
                device_id=(my ^ mask,), device_id_type=pl.DeviceIdType.MESH,
            )
        pl.semaphore_wait(barrier, len(MASKS))

        wins = (win0, win1, win2)
        wouts = (wout0, wout1, wout2)

        x_val = x_ref[...]
        for l in range(3):
            h = jnp.maximum(
                jnp.dot(x_val, wins[l][...], preferred_element_type=jnp.float32),
                0.0,
            )
            acc = jnp.dot(h, wouts[l][...], preferred_element_type=jnp.float32)
            chunks = [acc[OFFS[c]:OFFS[c] + CHUNKS[c], :] for c in range(3)]
            for r in range(3):
                rdmas = []
                for c in range(3):
                    mask = MASKS[(r + c) % 3]
                    sbufs[c][l, r] = chunks[c]
                    rdma = pltpu.make_async_remote_copy(
                        src_ref=sbufs[c].at[l, r],
                        dst_ref=rbufs[c].at[l, r],
                        send_sem=send_sems.at[l, r, c],
                        recv_sem=recv_sems.at[l, r, c],
                        device_id=(my ^ mask,),
                        device_id_type=pl.DeviceIdType.MESH,
                    )
                    rdma.start()
                    rdmas.append(rdma)
                for c in range(3):
                    rdmas[c].wait()
                    chunks[c] = chunks[c] + rbufs[c][l, r]
            x_val = jnp.concatenate(chunks, axis=0)
        out_ref[...] = x_val

    return pl.pallas_call(
        body,
        out_shape=jax.ShapeDtypeStruct((b, d), jnp.float32),
        in_specs=[pl.BlockSpec(memory_space=pltpu.VMEM)] * 7,
        out_specs=pl.BlockSpec(memory_space=pltpu.VMEM),
        scratch_shapes=[
            pltpu.VMEM((3, 3, CHUNKS[0], d), jnp.float32),
            pltpu.VMEM((3, 3, CHUNKS[1], d), jnp.float32),
            pltpu.VMEM((3, 3, CHUNKS[2], d), jnp.float32),
            pltpu.VMEM((3, 3, CHUNKS[0], d), jnp.float32),
            pltpu.VMEM((3, 3, CHUNKS[1], d), jnp.float32),
            pltpu.VMEM((3, 3, CHUNKS[2], d), jnp.float32),
            pltpu.SemaphoreType.DMA((3, 3, 3)),
            pltpu.SemaphoreType.DMA((3, 3, 3)),
        ],
        compiler_params=pltpu.CompilerParams(collective_id=0),
    )(x, Win0, Wout0, Win1, Wout1, Win2, Wout2)


# device time: 39360 ns/iter; 2.0172x vs baseline; 1.1888x over previous
import jax
import jax.numpy as jnp
from jax import lax
from jax.experimental import pallas as pl
from jax.experimental.pallas import tpu as pltpu

N_DEV = 8
MASKS = (1, 3, 4)
HB = 256
CHUNKS = (88, 84, 84)
OFFS = (0, 88, 172)


def kernel(x, Win0, Wout0, Win1, Wout1, Win2, Wout2):
    b, d = x.shape

    def body(x_ref, win0, wout0, win1, wout1, win2, wout2, out_ref,
             sb00, sb01, sb02, sb10, sb11, sb12,
             rb00, rb01, rb02, rb10, rb11, rb12,
             send_sems, recv_sems):
        my = lax.axis_index("i")
        sbufs = ((sb00, sb01, sb02), (sb10, sb11, sb12))
        rbufs = ((rb00, rb01, rb02), (rb10, rb11, rb12))

        barrier = pltpu.get_barrier_semaphore()
        for mask in MASKS:
            pl.semaphore_signal(
                barrier, inc=1,
                device_id=(my ^ mask,), device_id_type=pl.DeviceIdType.MESH,
            )
        pl.semaphore_wait(barrier, len(MASKS))

        wins = (win0, win1, win2)
        wouts = (wout0, wout1, wout2)

        def gemm(l, xh):
            h = jnp.maximum(
                jnp.dot(xh, wins[l][...], preferred_element_type=jnp.float32),
                0.0,
            )
            a = jnp.dot(h, wouts[l][...], preferred_element_type=jnp.float32)
            return [a[OFFS[c]:OFFS[c] + CHUNKS[c], :] for c in range(3)]

        def issue(l, hf, r, chunks):
            rdmas = []
            for c in range(3):
                mask = MASKS[(r + c) % 3]
                sbufs[hf][c][l, r] = chunks[c]
                rdma = pltpu.make_async_remote_copy(
                    src_ref=sbufs[hf][c].at[l, r],
                    dst_ref=rbufs[hf][c].at[l, r],
                    send_sem=send_sems.at[l, r, hf, c],
                    recv_sem=recv_sems.at[l, r, hf, c],
                    device_id=(my ^ mask,),
                    device_id_type=pl.DeviceIdType.MESH,
                )
                rdma.start()
                rdmas.append(rdma)
            return rdmas

        def finish(l, hf, r, rdmas, chunks):
            out = []
            for c in range(3):
                rdmas[c].wait()
                out.append(chunks[c] + rbufs[hf][c][l, r])
            return out

        ch = [None, None]
        rd = [None, None]
        ch[0] = gemm(0, x_ref[0:HB, :])
        rd[0] = issue(0, 0, 0, ch[0])
        ch[1] = gemm(0, x_ref[HB:2 * HB, :])
        rd[1] = issue(0, 1, 0, ch[1])

        for l in range(3):
            for r in (0, 1):
                ch[0] = finish(l, 0, r, rd[0], ch[0])
                rd[0] = issue(l, 0, r + 1, ch[0])
                ch[1] = finish(l, 1, r, rd[1], ch[1])
                rd[1] = issue(l, 1, r + 1, ch[1])
            for hf in (0, 1):
                ch[hf] = finish(l, hf, 2, rd[hf], ch[hf])
                if l < 2:
                    xh = jnp.concatenate(ch[hf], axis=0)
                    ch[hf] = gemm(l + 1, xh)
                    rd[hf] = issue(l + 1, hf, 0, ch[hf])

        for hf in (0, 1):
            for c in range(3):
                out_ref[hf * HB + OFFS[c]:hf * HB + OFFS[c] + CHUNKS[c], :] = \
                    ch[hf][c]

    return pl.pallas_call(
        body,
        out_shape=jax.ShapeDtypeStruct((b, d), jnp.float32),
        in_specs=[pl.BlockSpec(memory_space=pltpu.VMEM)] * 7,
        out_specs=pl.BlockSpec(memory_space=pltpu.VMEM),
        scratch_shapes=[
            pltpu.VMEM((3, 3, CHUNKS[c], d), jnp.float32)
            for hf in (0, 1) for c in range(3)
        ] + [
            pltpu.VMEM((3, 3, CHUNKS[c], d), jnp.float32)
            for hf in (0, 1) for c in range(3)
        ] + [
            pltpu.SemaphoreType.DMA((3, 3, 2, 3)),
            pltpu.SemaphoreType.DMA((3, 3, 2, 3)),
        ],
        compiler_params=pltpu.CompilerParams(collective_id=0),
    )(x, Win0, Wout0, Win1, Wout1, Win2, Wout2)


# device time: 36103 ns/iter; 2.1992x vs baseline; 1.0902x over previous
import jax
import jax.numpy as jnp
from jax import lax
from jax.experimental import pallas as pl
from jax.experimental.pallas import tpu as pltpu

N_DEV = 8
MASKS = (1, 3, 4)
S = 4
SB = 128
CHUNKS = (48, 40, 40)
OFFS = (0, 48, 88)


def kernel(x, Win0, Wout0, Win1, Wout1, Win2, Wout2):
    b, d = x.shape

    def body(*args):
        x_ref = args[0]
        wins = (args[1], args[3], args[5])
        wouts = (args[2], args[4], args[6])
        out_ref = args[7]
        sbufs = [[args[8 + s * 3 + c] for c in range(3)] for s in range(S)]
        rbufs = [[args[8 + 3 * S + s * 3 + c] for c in range(3)] for s in range(S)]
        send_sems = args[8 + 6 * S]
        recv_sems = args[9 + 6 * S]

        my = lax.axis_index("i")

        barrier = pltpu.get_barrier_semaphore()
        for mask in MASKS:
            pl.semaphore_signal(
                barrier, inc=1,
                device_id=(my ^ mask,), device_id_type=pl.DeviceIdType.MESH,
            )
        pl.semaphore_wait(barrier, len(MASKS))

        def gemm(l, xh):
            h = jnp.maximum(
                jnp.dot(xh, wins[l][...], preferred_element_type=jnp.float32),
                0.0,
            )
            a = jnp.dot(h, wouts[l][...], preferred_element_type=jnp.float32)
            return [a[OFFS[c]:OFFS[c] + CHUNKS[c], :] for c in range(3)]

        def make_rdma(l, s, r, c):
            return pltpu.make_async_remote_copy(
                src_ref=sbufs[s][c].at[l, r],
                dst_ref=rbufs[s][c].at[l, r],
                send_sem=send_sems.at[l, r, s, c],
                recv_sem=recv_sems.at[l, r, s, c],
                device_id=(my ^ MASKS[(r + c) % 3],),
                device_id_type=pl.DeviceIdType.MESH,
            )

        def issue(l, s, r, chunks):
            rdmas = []
            for c in range(3):
                sbufs[s][c][l, r] = chunks[c]
                rdma = make_rdma(l, s, r, c)
                rdma.start()
                rdmas.append(rdma)
            return rdmas

        def finish_issue(l, s, r, rdmas, chunks):
            new_rdmas = []
            new_chunks = []
            for c in range(3):
                rdmas[c].wait()
                v = chunks[c] + rbufs[s][c][l, r]
                sbufs[s][c][l, r + 1] = v
                rdma = make_rdma(l, s, r + 1, c)
                rdma.start()
                new_rdmas.append(rdma)
                new_chunks.append(v)
            return new_rdmas, new_chunks

        def finish_last(l, s, rdmas, chunks):
            out = []
            for c in range(3):
                rdmas[c].wait()
                out.append(chunks[c] + rbufs[s][c][l, 2])
            return out

        ch = [None] * S
        rd = [None] * S
        for s in range(S):
            ch[s] = gemm(0, x_ref[s * SB:(s + 1) * SB, :])
            rd[s] = issue(0, s, 0, ch[s])

        for l in range(3):
            for r in (0, 1):
                for s in range(S):
                    rd[s], ch[s] = finish_issue(l, s, r, rd[s], ch[s])
            for s in range(S):
                ch[s] = finish_last(l, s, rd[s], ch[s])
                if l < 2:
                    xh = jnp.concatenate(ch[s], axis=0)
                    ch[s] = gemm(l + 1, xh)
                    rd[s] = issue(l + 1, s, 0, ch[s])

        for s in range(S):
            for c in range(3):
                lo = s * SB + OFFS[c]
                out_ref[lo:lo + CHUNKS[c], :] = ch[s][c]

    return pl.pallas_call(
        body,
        out_shape=jax.ShapeDtypeStruct((b, d), jnp.float32),
        in_specs=[pl.BlockSpec(memory_space=pltpu.VMEM)] * 7,
        out_specs=pl.BlockSpec(memory_space=pltpu.VMEM),
        scratch_shapes=[
            pltpu.VMEM((3, 3, CHUNKS[c], d), jnp.float32)
            for _s in range(S) for c in range(3)
        ] + [
            pltpu.VMEM((3, 3, CHUNKS[c], d), jnp.float32)
            for _s in range(S) for c in range(3)
        ] + [
            pltpu.SemaphoreType.DMA((3, 3, S, 3)),
            pltpu.SemaphoreType.DMA((3, 3, S, 3)),
        ],
        compiler_params=pltpu.CompilerParams(collective_id=0),
    )(x, Win0, Wout0, Win1, Wout1, Win2, Wout2)
